# baseline (speedup 1.0000x reference)
import functools

import jax
import jax.numpy as jnp
from jax import lax
from jax.experimental import pallas as pl
from jax.experimental.pallas import tpu as pltpu
from jax.experimental.pallas import tpu_sc as plsc

SIG = 2048
LANES = 16
CH = 8


@functools.partial(jax.jit, static_argnums=(3,))
def _sc_interp(x, pp, tbl, B):
    info = plsc.get_sparse_core_info()
    nw = info.num_cores * info.num_subcores
    rows_per_w = B // nw
    n_chunks = rows_per_w // CH

    mesh = plsc.VectorSubcoreMesh(core_axis_name="c", subcore_axis_name="s")

    @functools.partial(
        pl.kernel,
        out_type=jax.ShapeDtypeStruct((B, SIG), jnp.float32),
        mesh=mesh,
        compiler_params=pltpu.CompilerParams(
            needs_layout_passes=False,
            use_tc_tiling_on_sc=True,
        ),
        scratch_types=[
            pltpu.VMEM((2, CH, SIG), jnp.float32),
            pltpu.VMEM((2, CH, SIG), jnp.float32),
            pltpu.VMEM((rows_per_w * LANES,), jnp.float32),
            pltpu.VMEM((5, SIG), jnp.float32),
            pltpu.SemaphoreType.DMA,
            pltpu.SemaphoreType.DMA,
            pltpu.SemaphoreType.DMA,
            pltpu.SemaphoreType.DMA,
        ],
    )
    def run(x_hbm, p_hbm, t_hbm, out_hbm, xbuf, obuf, pbuf, tbl_v,
            isem0, isem1, osem0, osem1):
        c = lax.axis_index("c")
        s = lax.axis_index("s")
        w = s * info.num_cores + c
        base = w * rows_per_w
        pltpu.sync_copy(t_hbm, tbl_v)
        pltpu.sync_copy(p_hbm.at[pl.ds(base * LANES, rows_per_w * LANES)], pbuf)

        lane_f = lax.iota(jnp.int32, LANES).astype(jnp.float32)
        zero16 = jnp.zeros((LANES,), jnp.int32)
        magic = jnp.float32(12582912.0)
        isems = (isem0, isem1)
        osems = (osem0, osem1)

        for b in range(2):
            pltpu.async_copy(
                x_hbm.at[pl.ds(base + b * CH, CH)], xbuf.at[b], isems[b]
            )

        def outer(g2, carry):
            for b in range(2):
                g = g2 * 2 + b
                row0 = base + g * CH
                xb = xbuf.at[b]
                ob = obuf.at[b]
                pltpu.make_async_copy(
                    x_hbm.at[pl.ds(row0, CH)], xb, isems[b]
                ).wait()
                @pl.when(g2 > 0)
                def _():
                    pltpu.make_async_copy(
                        ob, out_hbm.at[pl.ds(row0, CH)], osems[b]
                    ).wait()

                for r in range(0, CH, 2):
                    pbase_v = zero16 + (g * CH + r) * LANES
                    pa0 = plsc.load_gather(pbuf, [pbase_v])
                    pa1 = plsc.load_gather(pbuf, [pbase_v + 1])
                    pa2 = plsc.load_gather(pbuf, [pbase_v + 2])
                    pa3 = plsc.load_gather(pbuf, [pbase_v + 3])
                    pa4 = plsc.load_gather(pbuf, [pbase_v + 4])
                    pb0 = plsc.load_gather(pbuf, [pbase_v + LANES])
                    pb1 = plsc.load_gather(pbuf, [pbase_v + LANES + 1])
                    pb2 = plsc.load_gather(pbuf, [pbase_v + LANES + 2])
                    pb3 = plsc.load_gather(pbuf, [pbase_v + LANES + 3])
                    pb4 = plsc.load_gather(pbuf, [pbase_v + LANES + 4])
                    ra_v = zero16 + r
                    rb_v = zero16 + (r + 1)

                    @plsc.parallel_loop(0, SIG // LANES, unroll=3)
                    def vec_body(j):
                        col0 = j * LANES
                        b4 = tbl_v[0, pl.ds(col0, LANES)]
                        b3 = tbl_v[1, pl.ds(col0, LANES)]
                        b2 = tbl_v[2, pl.ds(col0, LANES)]
                        b1 = tbl_v[3, pl.ds(col0, LANES)]
                        fi = lane_f + (col0).astype(jnp.float32)
                        ca = ((pa0 * b4 + pa1 * b3) + (pa2 * b2 + pa3 * b1)) + pa4
                        cb = ((pb0 * b4 + pb1 * b3) + (pb2 * b2 + pb3 * b1)) + pb4
                        cvia = (ca + magic) - magic
                        cvib = (cb + magic) - magic
                        ka = ca - cvia
                        kb = cb - cvib
                        posfa = jnp.minimum(jnp.maximum(fi - cvia, 1.0), 2047.0)
                        posfb = jnp.minimum(jnp.maximum(fi - cvib, 1.0), 2047.0)
                        posa = posfa.astype(jnp.int32)
                        posb = posfb.astype(jnp.int32)
                        xa1 = plsc.load_gather(xb, [ra_v, posa])
                        xa2 = plsc.load_gather(xb, [ra_v, posa - 1])
                        xb1 = plsc.load_gather(xb, [rb_v, posb])
                        xb2 = plsc.load_gather(xb, [rb_v, posb - 1])
                        oa = (1.0 - ka) * xa1 + ka * xa2
                        ob_ = (1.0 - kb) * xb1 + kb * xb2
                        ob[r, pl.ds(col0, LANES)] = oa
                        ob[r + 1, pl.ds(col0, LANES)] = ob_

                pltpu.async_copy(ob, out_hbm.at[pl.ds(row0, CH)], osems[b])

                @pl.when(g2 < n_chunks // 2 - 1)
                def _():
                    pltpu.async_copy(
                        x_hbm.at[pl.ds(row0 + 2 * CH, CH)], xb, isems[b]
                    )

            return carry

        lax.fori_loop(0, n_chunks // 2, outer, 0)

        for b in range(2):
            pltpu.make_async_copy(
                obuf.at[b], out_hbm.at[pl.ds(base, CH)], osems[b]
            ).wait()

    return run(x, pp, tbl)


def kernel(x, params):
    scaler = jnp.array([[1e12, 1e8, 1e4, 1.0, 10.0]], dtype=jnp.float32)
    p = params / scaler
    pb = p.astype(jnp.bfloat16).astype(jnp.float32)
    pp = jnp.pad(pb, ((0, 0), (0, LANES - 5))).reshape(-1)
    si = jnp.arange(0, SIG, dtype=jnp.float32)
    tbl_bf = (
        jnp.stack([si**4, si**3, si**2, si], axis=0)
        .astype(jnp.bfloat16)
        .astype(jnp.float32)
    )
    tbl = jnp.concatenate([tbl_bf, si[None, :]], axis=0)
    return _sc_interp(x, pp, tbl, x.shape[0])


# speedup vs baseline: 1.0157x; 1.0157x over previous
import functools

import jax
import jax.numpy as jnp
from jax import lax
from jax.experimental import pallas as pl
from jax.experimental.pallas import tpu as pltpu
from jax.experimental.pallas import tpu_sc as plsc

SIG = 2048
LANES = 16
CH = 8


@functools.partial(jax.jit, static_argnums=(3,))
def _sc_interp(x, pp, tbl, B):
    info = plsc.get_sparse_core_info()
    nw = info.num_cores * info.num_subcores
    rows_per_w = B // nw
    n_chunks = rows_per_w // CH

    mesh = plsc.VectorSubcoreMesh(core_axis_name="c", subcore_axis_name="s")

    @functools.partial(
        pl.kernel,
        out_type=jax.ShapeDtypeStruct((B, SIG), jnp.float32),
        mesh=mesh,
        compiler_params=pltpu.CompilerParams(
            needs_layout_passes=False,
            use_tc_tiling_on_sc=True,
        ),
        scratch_types=[
            pltpu.VMEM((2, CH, SIG), jnp.float32),
            pltpu.VMEM((2, CH, SIG), jnp.float32),
            pltpu.VMEM((rows_per_w * LANES,), jnp.float32),
            pltpu.VMEM((5, SIG), jnp.float32),
            pltpu.SemaphoreType.DMA,
            pltpu.SemaphoreType.DMA,
            pltpu.SemaphoreType.DMA,
            pltpu.SemaphoreType.DMA,
        ],
    )
    def run(x_hbm, p_hbm, t_hbm, out_hbm, xbuf, obuf, pbuf, tbl_v,
            isem0, isem1, osem0, osem1):
        c = lax.axis_index("c")
        s = lax.axis_index("s")
        w = s * info.num_cores + c
        base = w * rows_per_w
        pltpu.sync_copy(t_hbm, tbl_v)
        pltpu.sync_copy(p_hbm.at[pl.ds(base * LANES, rows_per_w * LANES)], pbuf)

        lane_f = lax.iota(jnp.int32, LANES).astype(jnp.float32)
        zero16 = jnp.zeros((LANES,), jnp.int32)
        magic = jnp.float32(12582912.0)
        isems = (isem0, isem1)
        osems = (osem0, osem1)

        for b in range(2):
            pltpu.async_copy(
                x_hbm.at[pl.ds(base + b * CH, CH)], xbuf.at[b], isems[b]
            )

        def outer(g2, carry):
            for b in range(2):
                g = g2 * 2 + b
                row0 = base + g * CH
                xb = xbuf.at[b]
                ob = obuf.at[b]
                pltpu.make_async_copy(
                    x_hbm.at[pl.ds(row0, CH)], xb, isems[b]
                ).wait()
                @pl.when(g2 > 0)
                def _():
                    pltpu.make_async_copy(
                        ob, out_hbm.at[pl.ds(row0, CH)], osems[b]
                    ).wait()

                for r in range(0, CH, 2):
                    pbase_v = zero16 + (g * CH + r) * LANES
                    pa0 = plsc.load_gather(pbuf, [pbase_v])
                    pa1 = plsc.load_gather(pbuf, [pbase_v + 1])
                    pa2 = plsc.load_gather(pbuf, [pbase_v + 2])
                    pa3 = plsc.load_gather(pbuf, [pbase_v + 3])
                    pa4 = plsc.load_gather(pbuf, [pbase_v + 4])
                    pb0 = plsc.load_gather(pbuf, [pbase_v + LANES])
                    pb1 = plsc.load_gather(pbuf, [pbase_v + LANES + 1])
                    pb2 = plsc.load_gather(pbuf, [pbase_v + LANES + 2])
                    pb3 = plsc.load_gather(pbuf, [pbase_v + LANES + 3])
                    pb4 = plsc.load_gather(pbuf, [pbase_v + LANES + 4])
                    ra_v = zero16 + r
                    rb_v = zero16 + (r + 1)

                    @plsc.parallel_loop(0, SIG // LANES, unroll=2)
                    def vec_body(j):
                        col0 = j * LANES
                        b4 = tbl_v[0, pl.ds(col0, LANES)]
                        b3 = tbl_v[1, pl.ds(col0, LANES)]
                        b2 = tbl_v[2, pl.ds(col0, LANES)]
                        b1 = tbl_v[3, pl.ds(col0, LANES)]
                        fi = lane_f + (col0).astype(jnp.float32)
                        ca = ((pa0 * b4 + pa1 * b3) + (pa2 * b2 + pa3 * b1)) + pa4
                        cb = ((pb0 * b4 + pb1 * b3) + (pb2 * b2 + pb3 * b1)) + pb4
                        cvia = (ca + magic) - magic
                        cvib = (cb + magic) - magic
                        ka = ca - cvia
                        kb = cb - cvib
                        posfa = jnp.minimum(jnp.maximum(fi - cvia, 1.0), 2047.0)
                        posfb = jnp.minimum(jnp.maximum(fi - cvib, 1.0), 2047.0)
                        posa = posfa.astype(jnp.int32)
                        posb = posfb.astype(jnp.int32)
                        xa1 = plsc.load_gather(xb, [ra_v, posa])
                        xa2 = plsc.load_gather(xb, [ra_v, posa - 1])
                        xb1 = plsc.load_gather(xb, [rb_v, posb])
                        xb2 = plsc.load_gather(xb, [rb_v, posb - 1])
                        oa = (1.0 - ka) * xa1 + ka * xa2
                        ob_ = (1.0 - kb) * xb1 + kb * xb2
                        ob[r, pl.ds(col0, LANES)] = oa
                        ob[r + 1, pl.ds(col0, LANES)] = ob_

                pltpu.async_copy(ob, out_hbm.at[pl.ds(row0, CH)], osems[b])

                @pl.when(g2 < n_chunks // 2 - 1)
                def _():
                    pltpu.async_copy(
                        x_hbm.at[pl.ds(row0 + 2 * CH, CH)], xb, isems[b]
                    )

            return carry

        lax.fori_loop(0, n_chunks // 2, outer, 0)

        for b in range(2):
            pltpu.make_async_copy(
                obuf.at[b], out_hbm.at[pl.ds(base, CH)], osems[b]
            ).wait()

    return run(x, pp, tbl)


def kernel(x, params):
    scaler = jnp.array([[1e12, 1e8, 1e4, 1.0, 10.0]], dtype=jnp.float32)
    p = params / scaler
    pb = p.astype(jnp.bfloat16).astype(jnp.float32)
    pp = jnp.pad(pb, ((0, 0), (0, LANES - 5))).reshape(-1)
    si = jnp.arange(0, SIG, dtype=jnp.float32)
    tbl_bf = (
        jnp.stack([si**4, si**3, si**2, si], axis=0)
        .astype(jnp.bfloat16)
        .astype(jnp.float32)
    )
    tbl = jnp.concatenate([tbl_bf, si[None, :]], axis=0)
    return _sc_interp(x, pp, tbl, x.shape[0])
